# baseline (speedup 1.0000x reference)
import functools

import jax
import jax.numpy as jnp
from jax import lax
from jax.experimental import pallas as pl
from jax.experimental.pallas import tpu as pltpu
from jax.experimental.pallas import tpu_sc as plsc

_N = 10000
_E = 320000
_D = 128
_O = 40
_NPAD = 10240

_NC = 2
_NS = 16
_NW = _NC * _NS
_CH = 128
_WJ = 16
_NWIN = 5
_NCH = _WJ * _NWIN
_EPT = _CH * _NCH
_EPAD = _NW * _EPT
_RPT = _NPAD // _NS
_NRC = _RPT // _CH

_mesh = plsc.VectorSubcoreMesh(core_axis_name="c", subcore_axis_name="s")


@functools.partial(
    pl.kernel,
    mesh=_mesh,
    out_type=jax.ShapeDtypeStruct((_NC, _NPAD, _D), jnp.float32),
    scratch_types=[
        pltpu.VMEM((_WJ, _CH), jnp.int32),
        pltpu.VMEM((_WJ, _CH), jnp.int32),
        pltpu.VMEM((_CH, _D), jnp.float32),
        pltpu.VMEM((_CH, _D), jnp.float32),
        pltpu.VMEM_SHARED((_NPAD, _D), jnp.float32),
        pltpu.SemaphoreType.DMA,
        pltpu.SemaphoreType.DMA,
    ],
)
def _agg(srcr, dstr, hs, zhbm, out, sidx, didx, rows_a, rows_b, acc, sem_g, sem_s):
    c = lax.axis_index("c")
    s = lax.axis_index("s")
    w = s * _NC + c
    base = s * _RPT
    pltpu.sync_copy(zhbm.at[pl.ds(base, _RPT)], acc.at[pl.ds(base, _RPT)])
    plsc.subcore_barrier()
    bufs = (rows_a, rows_b)

    def win_body(win, carry):
        pltpu.sync_copy(srcr.at[w, win], sidx)
        pltpu.sync_copy(dstr.at[w, win], didx)
        dg = pltpu.async_copy(hs.at[sidx.at[0]], bufs[0], sem_g)
        ds_prev = None
        for jj in range(_WJ):
            dg.wait()
            ds = pltpu.async_copy(bufs[jj % 2], acc.at[didx.at[jj]], sem_s, add=True)
            if ds_prev is not None:
                ds_prev.wait()
            ds_prev = ds
            if jj + 1 < _WJ:
                dg = pltpu.async_copy(hs.at[sidx.at[jj + 1]], bufs[(jj + 1) % 2], sem_g)
        ds_prev.wait()
        return carry

    lax.fori_loop(0, _NWIN, win_body, 0)
    plsc.subcore_barrier()
    pltpu.sync_copy(acc.at[pl.ds(base, _RPT)], out.at[c, pl.ds(base, _RPT)])


@functools.partial(
    pl.kernel,
    mesh=_mesh,
    out_type=jax.ShapeDtypeStruct((_NC, _NPAD), jnp.float32),
    scratch_types=[
        pltpu.VMEM((_NCH, _CH), jnp.int32),
        pltpu.VMEM((_CH,), jnp.float32),
        pltpu.VMEM((_RPT,), jnp.float32),
        pltpu.VMEM_SHARED((_NPAD,), jnp.float32),
        pltpu.SemaphoreType.DMA,
    ],
)
def _deg_kernel(dstr, ones_hbm, zrow_hbm, out, didx, ones_v, dbuf, accd, sem):
    c = lax.axis_index("c")
    s = lax.axis_index("s")
    w = s * _NC + c
    base = s * _RPT
    pltpu.sync_copy(zrow_hbm, dbuf)
    pltpu.sync_copy(dbuf, accd.at[pl.ds(base, _RPT)])
    pltpu.sync_copy(ones_hbm, ones_v)
    pltpu.sync_copy(dstr.at[w], didx)
    plsc.subcore_barrier()

    def body(j, carry):
        pltpu.sync_copy(ones_v, accd.at[didx.at[j]], add=True)
        return carry

    lax.fori_loop(0, _NCH, body, 0)
    plsc.subcore_barrier()
    pltpu.sync_copy(accd.at[pl.ds(base, _RPT)], dbuf)
    pltpu.sync_copy(dbuf, out.at[c, pl.ds(base, _RPT)])


def _dot_t(a, w):
    return lax.dot_general(
        a, w, (((1,), (1,)), ((), ())),
        preferred_element_type=jnp.float32,
        precision=lax.Precision.HIGHEST,
    )


def _valid_rows():
    return lax.broadcasted_iota(jnp.int32, (_NPAD, 1), 0) < _N


def _tc_pre(degT, xp, W1):
    def body(deg_ref, x_ref, w_ref, hs_ref, dinv_ref):
        dsum = deg_ref[:, 0:1] + deg_ref[:, 1:2] + 1.0
        dinv = lax.rsqrt(dsum)
        hs_ref[...] = _dot_t(x_ref[...], w_ref[...]) * dinv
        dinv_ref[...] = dinv

    return pl.pallas_call(
        body,
        out_shape=(
            jax.ShapeDtypeStruct((_NPAD, _D), jnp.float32),
            jax.ShapeDtypeStruct((_NPAD, 1), jnp.float32),
        ),
    )(degT, xp, W1)


def _tc_mid(acc, hs, dinv, b, g, beta, Wn):
    wout = Wn.shape[0]

    def body(acc_ref, hs_ref, dinv_ref, b_ref, g_ref, beta_ref, w_ref, out_ref):
        dinv = dinv_ref[...]
        y = (hs_ref[...] + acc_ref[0] + acc_ref[1]) * dinv + b_ref[...]
        valid = _valid_rows()
        ym = jnp.where(valid, y, 0.0)
        mean = jnp.sum(ym, axis=0, keepdims=True) * (1.0 / _N)
        d = y - mean
        var = jnp.sum(jnp.where(valid, d * d, 0.0), axis=0, keepdims=True) * (1.0 / _N)
        z = jnp.maximum(g_ref[...] * d * lax.rsqrt(var + 1e-5) + beta_ref[...], 0.0)
        hs_next = _dot_t(z, w_ref[...]) * dinv
        out_ref[...] = jnp.where(valid, hs_next, 0.0)

    return pl.pallas_call(
        body,
        out_shape=jax.ShapeDtypeStruct((_NPAD, wout), jnp.float32),
    )(acc, hs, dinv, b, g, beta, Wn)


def _tc_post(acc, hs, dinv, b):
    def body(acc_ref, hs_ref, dinv_ref, b_ref, out_ref):
        out_ref[...] = (hs_ref[...] + acc_ref[0] + acc_ref[1]) * dinv_ref[...] + b_ref[...]

    return pl.pallas_call(
        body,
        out_shape=jax.ShapeDtypeStruct((_NPAD, _D), jnp.float32),
    )(acc, hs, dinv, b)


def kernel(x, edge_index, W1, b1, g1, beta1, W2, b2, g2, beta2, W3, b3):
    npad_extra = _EPAD - _E
    fill = _N + (jnp.arange(npad_extra, dtype=jnp.int32) % (_NPAD - _N))
    src_p = jnp.concatenate([edge_index[0], fill])
    dst_p = jnp.concatenate([edge_index[1], fill])
    srcr = src_p.reshape(_NW, _NWIN, _WJ, _CH)
    dstr = dst_p.reshape(_NW, _NWIN, _WJ, _CH)
    dstr_deg = dst_p.reshape(_NW, _NCH, _CH)
    xp = jnp.pad(x, ((0, _NPAD - _N), (0, 0)))
    zhbm = jnp.zeros((_NPAD, _D), jnp.float32)
    zrow = jnp.zeros((_RPT,), jnp.float32)
    ones_ch = jnp.ones((_CH,), jnp.float32)

    deg2 = _deg_kernel(dstr_deg, ones_ch, zrow)
    hs1, dinv = _tc_pre(deg2.T, xp, W1)
    acc1 = _agg(srcr, dstr, hs1, zhbm)
    hs2 = _tc_mid(acc1, hs1, dinv, b1[None], g1[None], beta1[None], W2)
    acc2 = _agg(srcr, dstr, hs2, zhbm)
    W3p = jnp.pad(W3, ((0, _D - _O), (0, 0)))
    b3p = jnp.pad(b3, (0, _D - _O))
    hs3 = _tc_mid(acc2, hs2, dinv, b2[None], g2[None], beta2[None], W3p)
    acc3 = _agg(srcr, dstr, hs3, zhbm)
    out = _tc_post(acc3, hs3, dinv, b3p[None])
    res = out[:_N, :_O]
    return (res, res)


# speedup vs baseline: 1.0432x; 1.0432x over previous
import functools

import jax
import jax.numpy as jnp
from jax import lax
from jax.experimental import pallas as pl
from jax.experimental.pallas import tpu as pltpu
from jax.experimental.pallas import tpu_sc as plsc

_N = 10000
_E = 320000
_D = 128
_O = 40
_NPAD = 10240

_NC = 2
_NS = 16
_NW = _NC * _NS
_CH = 128
_WJ = 16
_NWIN = 5
_NCH = _WJ * _NWIN
_EPT = _CH * _NCH
_EPAD = _NW * _EPT
_RPT = _NPAD // _NS
_NRC = _RPT // _CH

_mesh = plsc.VectorSubcoreMesh(core_axis_name="c", subcore_axis_name="s")


@functools.partial(
    pl.kernel,
    mesh=_mesh,
    out_type=jax.ShapeDtypeStruct((_NC, _NPAD, _D), jnp.float32),
    scratch_types=[
        pltpu.VMEM((_WJ, _CH), jnp.int32),
        pltpu.VMEM((_WJ, _CH), jnp.int32),
        pltpu.VMEM((_WJ, _CH), jnp.int32),
        pltpu.VMEM((_WJ, _CH), jnp.int32),
        pltpu.VMEM((_CH, _D), jnp.float32),
        pltpu.VMEM((_CH, _D), jnp.float32),
        pltpu.VMEM_SHARED((_NPAD, _D), jnp.float32),
        pltpu.SemaphoreType.DMA,
        pltpu.SemaphoreType.DMA,
        pltpu.SemaphoreType.DMA,
    ],
)
def _agg(srcr, dstr, hs, zhbm, out, sidx_a, sidx_b, didx_a, didx_b,
         rows_a, rows_b, acc, sem_g, sem_s, sem_i):
    c = lax.axis_index("c")
    s = lax.axis_index("s")
    w = s * _NC + c
    base = s * _RPT
    @pl.when(c == 0)
    def _():
        pltpu.sync_copy(hs.at[pl.ds(base, _RPT)], acc.at[pl.ds(base, _RPT)])

    @pl.when(c != 0)
    def _():
        pltpu.sync_copy(zhbm.at[pl.ds(base, _RPT)], acc.at[pl.ds(base, _RPT)])

    plsc.subcore_barrier()
    bufs = (rows_a, rows_b)
    sidxs = (sidx_a, sidx_b)
    didxs = (didx_a, didx_b)
    pltpu.sync_copy(srcr.at[w, 0], sidx_a)
    pltpu.sync_copy(dstr.at[w, 0], didx_a)
    di_prev = None
    for win in range(_NWIN):
        sidx = sidxs[win % 2]
        didx = didxs[win % 2]
        if win + 1 < _NWIN:
            di_s = pltpu.async_copy(srcr.at[w, win + 1], sidxs[(win + 1) % 2], sem_i)
            di_d = pltpu.async_copy(dstr.at[w, win + 1], didxs[(win + 1) % 2], sem_i)
            di_prev = (di_s, di_d)
        dg = pltpu.async_copy(hs.at[sidx.at[0]], bufs[0], sem_g)
        ds_prev = None
        for jj in range(_WJ):
            dg.wait()
            ds = pltpu.async_copy(bufs[jj % 2], acc.at[didx.at[jj]], sem_s, add=True)
            if ds_prev is not None:
                ds_prev.wait()
            ds_prev = ds
            if jj + 1 < _WJ:
                dg = pltpu.async_copy(hs.at[sidx.at[jj + 1]], bufs[(jj + 1) % 2], sem_g)
        ds_prev.wait()
        if win + 1 < _NWIN:
            di_prev[0].wait()
            di_prev[1].wait()
    plsc.subcore_barrier()
    pltpu.sync_copy(acc.at[pl.ds(base, _RPT)], out.at[c, pl.ds(base, _RPT)])


@functools.partial(
    pl.kernel,
    mesh=_mesh,
    out_type=jax.ShapeDtypeStruct((_NC, _NPAD), jnp.float32),
    scratch_types=[
        pltpu.VMEM((_NCH, _CH), jnp.int32),
        pltpu.VMEM((_CH,), jnp.float32),
        pltpu.VMEM((_RPT,), jnp.float32),
        pltpu.VMEM_SHARED((_NPAD,), jnp.float32),
        pltpu.SemaphoreType.DMA,
    ],
)
def _deg_kernel(dstr, ones_hbm, zrow_hbm, out, didx, ones_v, dbuf, accd, sem):
    c = lax.axis_index("c")
    s = lax.axis_index("s")
    w = s * _NC + c
    base = s * _RPT
    pltpu.sync_copy(zrow_hbm, dbuf)
    pltpu.sync_copy(dbuf, accd.at[pl.ds(base, _RPT)])
    pltpu.sync_copy(ones_hbm, ones_v)
    pltpu.sync_copy(dstr.at[w], didx)
    plsc.subcore_barrier()

    def body(j, carry):
        pltpu.sync_copy(ones_v, accd.at[didx.at[j]], add=True)
        return carry

    lax.fori_loop(0, _NCH, body, 0)
    plsc.subcore_barrier()
    pltpu.sync_copy(accd.at[pl.ds(base, _RPT)], dbuf)
    pltpu.sync_copy(dbuf, out.at[c, pl.ds(base, _RPT)])


def _dot_t(a, w):
    return lax.dot_general(
        a, w, (((1,), (1,)), ((), ())),
        preferred_element_type=jnp.float32,
    )


def _valid_rows():
    return lax.broadcasted_iota(jnp.int32, (_NPAD, 1), 0) < _N


def _tc_pre(degT, xp, W1):
    def body(deg_ref, x_ref, w_ref, hs_ref, dinv_ref):
        dsum = deg_ref[:, 0:1] + deg_ref[:, 1:2] + 1.0
        dinv = lax.rsqrt(dsum)
        hs_ref[...] = _dot_t(x_ref[...], w_ref[...]) * dinv
        dinv_ref[...] = dinv

    return pl.pallas_call(
        body,
        out_shape=(
            jax.ShapeDtypeStruct((_NPAD, _D), jnp.float32),
            jax.ShapeDtypeStruct((_NPAD, 1), jnp.float32),
        ),
    )(degT, xp, W1)


def _tc_mid(acc, dinv, b, g, beta, Wn):
    wout = Wn.shape[0]

    def body(acc_ref, dinv_ref, b_ref, g_ref, beta_ref, w_ref, out_ref):
        dinv = dinv_ref[...]
        y = (acc_ref[0] + acc_ref[1]) * dinv + b_ref[...]
        valid = _valid_rows()
        ym = jnp.where(valid, y, 0.0)
        mean = jnp.sum(ym, axis=0, keepdims=True) * (1.0 / _N)
        d = y - mean
        var = jnp.sum(jnp.where(valid, d * d, 0.0), axis=0, keepdims=True) * (1.0 / _N)
        z = jnp.maximum(g_ref[...] * d * lax.rsqrt(var + 1e-5) + beta_ref[...], 0.0)
        hs_next = _dot_t(z, w_ref[...]) * dinv
        out_ref[...] = jnp.where(valid, hs_next, 0.0)

    return pl.pallas_call(
        body,
        out_shape=jax.ShapeDtypeStruct((_NPAD, wout), jnp.float32),
    )(acc, dinv, b, g, beta, Wn)


def _tc_post(acc, dinv, b):
    def body(acc_ref, dinv_ref, b_ref, out_ref):
        out_ref[...] = (acc_ref[0] + acc_ref[1]) * dinv_ref[...] + b_ref[...]

    return pl.pallas_call(
        body,
        out_shape=jax.ShapeDtypeStruct((_NPAD, _D), jnp.float32),
    )(acc, dinv, b)


def kernel(x, edge_index, W1, b1, g1, beta1, W2, b2, g2, beta2, W3, b3):
    npad_extra = _EPAD - _E
    fill = _N + (jnp.arange(npad_extra, dtype=jnp.int32) % (_NPAD - _N))
    src_p = jnp.concatenate([edge_index[0], fill])
    dst_p = jnp.concatenate([edge_index[1], fill])
    srcr = src_p.reshape(_NW, _NWIN, _WJ, _CH)
    dstr = dst_p.reshape(_NW, _NWIN, _WJ, _CH)
    dstr_deg = dst_p.reshape(_NW, _NCH, _CH)
    xp = jnp.pad(x, ((0, _NPAD - _N), (0, 0)))
    zhbm = jnp.zeros((_NPAD, _D), jnp.float32)
    zrow = jnp.zeros((_RPT,), jnp.float32)
    ones_ch = jnp.ones((_CH,), jnp.float32)

    deg2 = _deg_kernel(dstr_deg, ones_ch, zrow)
    hs1, dinv = _tc_pre(deg2.T, xp, W1)
    acc1 = _agg(srcr, dstr, hs1, zhbm)
    hs2 = _tc_mid(acc1, dinv, b1[None], g1[None], beta1[None], W2)
    acc2 = _agg(srcr, dstr, hs2, zhbm)
    W3p = jnp.pad(W3, ((0, _D - _O), (0, 0)))
    b3p = jnp.pad(b3, (0, _D - _O))
    hs3 = _tc_mid(acc2, dinv, b2[None], g2[None], beta2[None], W3p)
    acc3 = _agg(srcr, dstr, hs3, zhbm)
    out = _tc_post(acc3, dinv, b3p[None])
    res = out[:_N, :_O]
    return (res, res)
